# Initial kernel scaffold; baseline (speedup 1.0000x reference)
import jax
import jax.numpy as jnp
from jax.experimental import pallas as pl


def kernel(x, edge_index, edge_attr, batch, s_label, edge_W, W1, b1, W2, b2, gp_raw_ls, gp_raw_os, gp_raw_noise, gp_mean):
    raise NotImplementedError("write your pallas kernel here")


# speedup vs baseline: 3.1336x; 3.1336x over previous
import functools

import jax
import jax.numpy as jnp
from jax import lax
from jax.experimental import pallas as pl
from jax.experimental.pallas import tpu as pltpu
from jax.experimental.pallas import tpu_sc as plsc

_N = 10000
_E = 320000
_D = 128
_G = 128
_L = 5

_NC = 2
_NS = 16
_NW = _NC * _NS
_CH = 128
_NCHUNK = 79
_EPW = _CH * _NCHUNK
_EPAD = _EPW * _NW
_NPAD = 10240
_ZROWS = _NPAD // _NS

_HIGH = jax.lax.Precision.HIGHEST


def _sc_edge_scatter_body(gather, h_hbm, src_hbm, dst_hbm, z_hbm, out_hbm,
                          acc, src_v, dst_v, rows_v, sem):
    c = lax.axis_index("c")
    s = lax.axis_index("s")
    wid = s * _NC + c

    pltpu.sync_copy(z_hbm, acc.at[pl.ds(s * _ZROWS, _ZROWS)])
    plsc.subcore_barrier()

    base = wid * _EPW

    def body(i, carry):
        off = base + i * _CH
        pltpu.sync_copy(dst_hbm.at[pl.ds(off, _CH)], dst_v)
        if gather:
            pltpu.sync_copy(src_hbm.at[pl.ds(off, _CH)], src_v)
            pltpu.async_copy(h_hbm.at[src_v], rows_v, sem).wait()
        else:
            pltpu.sync_copy(h_hbm.at[pl.ds(off, _CH)], rows_v)
        pltpu.sync_copy(rows_v, acc.at[dst_v], add=True)
        return carry

    lax.fori_loop(0, _NCHUNK, body, 0)
    plsc.subcore_barrier()
    pltpu.sync_copy(acc.at[pl.ds(s * _ZROWS, _ZROWS)],
                    out_hbm.at[c, pl.ds(s * _ZROWS, _ZROWS)])


@functools.lru_cache(maxsize=None)
def _make_sc_scatter(width, gather):
    mesh = plsc.VectorSubcoreMesh(core_axis_name="c", subcore_axis_name="s")
    return pl.kernel(
        functools.partial(_sc_edge_scatter_body, gather),
        out_type=jax.ShapeDtypeStruct((_NC, _NPAD, width), jnp.float32),
        mesh=mesh,
        scratch_types=[
            pltpu.VMEM_SHARED((_NPAD, width), jnp.float32),
            pltpu.VMEM((_CH,), jnp.int32),
            pltpu.VMEM((_CH,), jnp.int32),
            pltpu.VMEM((_CH, width), jnp.float32),
            pltpu.SemaphoreType.DMA,
        ],
    )


def _sc_scatter_h(h, srcp, dstp, z):
    return _make_sc_scatter(_D, True)(h, srcp, dstp, z)


def _sc_scatter_a(attrp, srcp, dstp, z):
    return _make_sc_scatter(16, False)(attrp, srcp, dstp, z)


_RB = 1000
_NB = _N // _RB


def _dense_body(relu_out, h_ref, agg_ref, ap_ref, ew_ref, w1_ref, b1_ref,
                w2_ref, b2_ref, o_ref):
    a = ap_ref[0] + ap_ref[1]
    c = jnp.dot(a, ew_ref[...], preferred_element_type=jnp.float32,
                precision=_HIGH)
    pre = h_ref[...] + agg_ref[0] + agg_ref[1] + c
    hid = jnp.dot(pre, w1_ref[...], preferred_element_type=jnp.float32,
                  precision=_HIGH) + b1_ref[...]
    hid = jnp.maximum(hid, 0.0)
    out = jnp.dot(hid, w2_ref[...], preferred_element_type=jnp.float32,
                  precision=_HIGH) + b2_ref[...]
    if relu_out:
        out = jnp.maximum(out, 0.0)
    o_ref[...] = out


def _make_dense(relu_out):
    return pl.pallas_call(
        functools.partial(_dense_body, relu_out),
        grid=(_NB,),
        in_specs=[
            pl.BlockSpec((_RB, _D), lambda g: (g, 0)),
            pl.BlockSpec((_NC, _RB, _D), lambda g: (0, g, 0)),
            pl.BlockSpec((_NC, _RB, 16), lambda g: (0, g, 0)),
            pl.BlockSpec((16, _D), lambda g: (0, 0)),
            pl.BlockSpec((_D, 2 * _D), lambda g: (0, 0)),
            pl.BlockSpec((1, 2 * _D), lambda g: (0, 0)),
            pl.BlockSpec((2 * _D, _D), lambda g: (0, 0)),
            pl.BlockSpec((1, _D), lambda g: (0, 0)),
        ],
        out_specs=pl.BlockSpec((_RB, _D), lambda g: (g, 0)),
        out_shape=jax.ShapeDtypeStruct((_N, _D), jnp.float32),
    )


_dense_mid = _make_dense(True)
_dense_last = _make_dense(False)


def _pool_body(batch_ref, h_ref, sums_ref, cnt_ref):
    g = pl.program_id(0)

    @pl.when(g == 0)
    def _init():
        sums_ref[...] = jnp.zeros_like(sums_ref)
        cnt_ref[...] = jnp.zeros_like(cnt_ref)

    b = batch_ref[0]
    gid = lax.broadcasted_iota(jnp.int32, (_G, 1), 0)
    oh = (b == gid).astype(jnp.float32)
    sums_ref[...] += jnp.dot(oh, h_ref[...],
                             preferred_element_type=jnp.float32,
                             precision=_HIGH)
    cnt = jnp.sum(oh, axis=1, keepdims=True)
    cnt_ref[...] += jnp.broadcast_to(cnt, (_G, _D))


_pool = pl.pallas_call(
    _pool_body,
    grid=(_NB,),
    in_specs=[
        pl.BlockSpec((1, 1, _RB), lambda g: (g, 0, 0)),
        pl.BlockSpec((_RB, _D), lambda g: (g, 0)),
    ],
    out_specs=[
        pl.BlockSpec((_G, _D), lambda g: (0, 0)),
        pl.BlockSpec((_G, _D), lambda g: (0, 0)),
    ],
    out_shape=[
        jax.ShapeDtypeStruct((_G, _D), jnp.float32),
        jax.ShapeDtypeStruct((_G, _D), jnp.float32),
    ],
)


def _softplus(x):
    return jnp.maximum(x, 0.0) + jnp.log1p(jnp.exp(-jnp.abs(x)))


def _colget(M, ej):
    return lax.dot_general(ej, M, (((1,), (1,)), ((), ())), precision=_HIGH)


def _rowget(M, ej):
    return lax.dot_general(ej, M, (((1,), (0,)), ((), ())), precision=_HIGH)


def _outer(u, v):
    return lax.dot_general(u, v, (((0,), (0,)), ((), ())), precision=_HIGH)


def _head_body(sums_ref, cnt_ref, s01_ref, rls_ref, ros_ref, rnz_ref,
               rmn_ref, o_ref):
    ls = _softplus(rls_ref[0, 0])
    os_ = _softplus(ros_ref[0, 0])
    noise = _softplus(rnz_ref[0, 0])
    mean_c = rmn_ref[0, 0]

    cnt = cnt_ref[:, 0:1]
    feat = sums_ref[...] / jnp.maximum(cnt, 1.0)
    f = feat / ls

    ff = f * f
    sq_col = jnp.sum(ff, axis=1, keepdims=True)
    ones_r = jnp.ones((1, _G), jnp.float32)
    sq_row = lax.dot_general(ones_r, ff, (((1,), (1,)), ((), ())),
                             precision=_HIGH)
    gram = lax.dot_general(f, f, (((1,), (1,)), ((), ())),
                           precision=_HIGH)
    d2 = jnp.maximum(sq_col + sq_row - 2.0 * gram, 0.0)
    d = jnp.sqrt(d2 + 1e-12)
    s5d = jnp.sqrt(jnp.float32(5.0)) * d

    r_iota = lax.broadcasted_iota(jnp.int32, (_G, _G), 0)
    c_iota = lax.broadcasted_iota(jnp.int32, (_G, _G), 1)
    eye = (r_iota == c_iota).astype(jnp.float32)

    K = os_ * (1.0 + s5d + (5.0 / 3.0) * d2) * jnp.exp(-s5d)
    K = K + (noise + 1e-6) * eye

    lane = lax.broadcasted_iota(jnp.int32, (1, _G), 1)

    def chol_step(j, carry):
        M, Lm = carry
        ej = (lane == j).astype(jnp.float32)
        colj = _colget(M, ej)
        piv = jnp.sum(colj * ej)
        cvec = jnp.where(lane >= j, colj, 0.0) / jnp.sqrt(piv)
        Lm = Lm + _outer(cvec, ej)
        M = M - _outer(cvec, cvec)
        return M, Lm

    _, Lm = lax.fori_loop(0, _G, chol_step,
                          (K, jnp.zeros((_G, _G), jnp.float32)))

    resid = (s01_ref[...] - 0.5) * 2.0 - mean_c

    def fwd_step(j, z):
        ej = (lane == j).astype(jnp.float32)
        rowj = _rowget(Lm, ej)
        ljj = jnp.sum(rowj * ej)
        rj = jnp.sum(resid * ej)
        dotv = jnp.sum(rowj * z)
        return z + ej * ((rj - dotv) / ljj)

    z = lax.fori_loop(0, _G, fwd_step, jnp.zeros((1, _G), jnp.float32))

    def bwd_step(t, w):
        j = _G - 1 - t
        ej = (lane == j).astype(jnp.float32)
        colj = _colget(Lm, ej)
        ljj = jnp.sum(colj * ej)
        zj = jnp.sum(z * ej)
        dotv = jnp.sum(colj * w)
        return w + ej * ((zj - dotv) / ljj)

    w = lax.fori_loop(0, _G, bwd_step, jnp.zeros((1, _G), jnp.float32))

    quad = jnp.sum(resid * w)
    diag_row = jnp.sum(Lm * eye, axis=0, keepdims=True)
    logdet = jnp.sum(jnp.log(diag_row))
    gf = jnp.float32(_G)
    mll = -0.5 * quad - logdet - 0.5 * gf * jnp.log(2.0 * jnp.float32(jnp.pi))
    o_ref[...] = jnp.broadcast_to(-(mll / gf), (1, 1))


_head = pl.pallas_call(
    _head_body,
    out_shape=jax.ShapeDtypeStruct((1, 1), jnp.float32),
)


@jax.jit
def kernel(x, edge_index, edge_attr, batch, s_label, edge_W, W1, b1, W2, b2,
           gp_raw_ls, gp_raw_os, gp_raw_noise, gp_mean):
    pad = _EPAD - _E
    srcp = jnp.concatenate([edge_index[0],
                            jnp.zeros((pad,), jnp.int32)])
    dstp = jnp.concatenate([edge_index[1],
                            jnp.full((pad,), _N, jnp.int32)])
    attrp = jnp.concatenate(
        [jnp.concatenate([edge_attr,
                          jnp.zeros((_E, 12), jnp.float32)], axis=1),
         jnp.zeros((pad, 16), jnp.float32)], axis=0)

    z128 = jnp.zeros((_ZROWS, _D), jnp.float32)
    z16 = jnp.zeros((_ZROWS, 16), jnp.float32)

    a_part = _sc_scatter_a(attrp, srcp, dstp, z16)

    ewp = jnp.concatenate([edge_W, jnp.zeros((_L, 12, _D), jnp.float32)],
                          axis=1)
    b1r = b1.reshape(_L, 1, 2 * _D)
    b2r = b2.reshape(_L, 1, _D)

    h = x
    for l in range(_L):
        agg = _sc_scatter_h(h, srcp, dstp, z128)
        dense = _dense_mid if l < _L - 1 else _dense_last
        h = dense(h, agg, a_part, ewp[l], W1[l], b1r[l], W2[l], b2r[l])

    batch_r = batch.reshape(_NB, 1, _RB)
    sums, cntb = _pool(batch_r, h)

    s01 = s_label.astype(jnp.float32).reshape(1, _G)
    out = _head(sums, cntb, s01,
                jnp.reshape(gp_raw_ls, (1, 1)),
                jnp.reshape(gp_raw_os, (1, 1)),
                jnp.reshape(gp_raw_noise, (1, 1)),
                jnp.reshape(gp_mean, (1, 1)))
    return out[0, 0]
